# Initial kernel scaffold; baseline (speedup 1.0000x reference)
import jax
import jax.numpy as jnp
from jax.experimental import pallas as pl


def kernel(distances_uv, vectors_uv, atomic_charges, atomic_dipoles, idx_u, idx_v):
    raise NotImplementedError("write your pallas kernel here")


# speedup vs baseline: 200.3496x; 200.3496x over previous
import functools

import jax
import jax.numpy as jnp
from jax import lax
from jax.experimental import pallas as pl
from jax.experimental.pallas import tpu as pltpu
from jax.experimental.pallas import tpu_sc as plsc

CUTOFF = 10.0
CUTOFF_SHORT_RANGE = 2.0
KEHALF = 7.199822675975274
OFFSET2 = 1.0

NC = 2
NS = 16
L = 16
NW = NC * NS


def _rsqrt(s):
    yi = jnp.int32(0x5F3759DF) - (plsc.bitcast(s, jnp.int32) >> 1)
    y = plsc.bitcast(yi, jnp.float32)
    for _ in range(3):
        y = y * (1.5 - 0.5 * s * y * y)
    return y


def _edge_energy(d, vx, vy, vz, qu, qv, dux, duy, duz, dvx, dvy, dvz):
    x = d * (1.0 / CUTOFF_SHORT_RANGE)
    x2 = x * x
    x3 = x2 * x
    x4 = x3 * x
    x5 = x4 * x
    fx = 1.0 - 10.0 * x3 + 15.0 * x4 - 6.0 * x5
    sw = jnp.where(x < 1.0, fx, jnp.zeros_like(fx))
    inv_d = _rsqrt(d * d)
    inv_damped = _rsqrt(d * d + OFFSET2)
    chi = sw * inv_damped + (1.0 - sw) * inv_d
    chi2 = chi * chi
    chi3 = chi2 * chi
    cvx = vx * inv_d
    cvy = vy * inv_d
    cvz = vz * inv_d
    dot_uv = cvx * dvx + cvy * dvy + cvz * dvz
    dot_vu = cvx * dux + cvy * duy + cvz * duz
    dd = dux * dvx + duy * dvy + duz * dvz
    e = qu * qv * chi
    e = e + 2.0 * qu * dot_uv * chi2
    e = e + (dd - 3.0 * dot_uv * dot_vu) * chi3
    e = KEHALF * e
    return jnp.where(d <= CUTOFF, e, jnp.zeros_like(e))


def _make_sc_kernel(E, N, C):
    per_w = E // NW
    n_chunks = per_w // C
    n_pairs = n_chunks // 2
    n_vecs = C // L

    mesh = plsc.VectorSubcoreMesh(core_axis_name="c", subcore_axis_name="s")

    buf_types = [
        pltpu.VMEM((C,), jnp.int32),
        pltpu.VMEM((C,), jnp.int32),
        pltpu.VMEM((C, 4), jnp.float32),
        pltpu.VMEM((C, 4), jnp.float32),
        pltpu.VMEM((C,), jnp.float32),
        pltpu.VMEM((C,), jnp.float32),
        pltpu.VMEM((C,), jnp.float32),
        pltpu.VMEM((C,), jnp.float32),
        pltpu.VMEM((C,), jnp.float32),
        pltpu.SemaphoreType.DMA,
        pltpu.SemaphoreType.DMA,
        pltpu.SemaphoreType.DMA,
    ]

    @functools.partial(
        pl.kernel,
        out_type=jax.ShapeDtypeStruct((E,), jnp.float32),
        mesh=mesh,
        scratch_types=buf_types + buf_types,
        compiler_params=pltpu.CompilerParams(
            needs_layout_passes=False, use_tc_tiling_on_sc=False),
    )
    def sc_kernel(dist_hbm, vx_hbm, vy_hbm, vz_hbm, table_hbm,
                  idxu_hbm, idxv_hbm, out_hbm, *scratch):
        wid = lax.axis_index("s") * NC + lax.axis_index("c")
        base = wid * per_w
        nb = len(buf_types)
        bufs = [scratch[:nb], scratch[nb:]]

        def issue_chunk(j, buf):
            (idxu_v, idxv_v, rows_u, rows_v, dist_v,
             vx_v, vy_v, vz_v, out_v, sem_g, sem_l, sem_o) = buf
            cbase = base + j * C
            pltpu.sync_copy(idxu_hbm.at[pl.ds(cbase, C)], idxu_v)
            pltpu.sync_copy(idxv_hbm.at[pl.ds(cbase, C)], idxv_v)
            pltpu.async_copy(table_hbm.at[idxu_v], rows_u, sem_g)
            pltpu.async_copy(table_hbm.at[idxv_v], rows_v, sem_g)
            pltpu.async_copy(dist_hbm.at[pl.ds(cbase, C)], dist_v, sem_l)
            pltpu.async_copy(vx_hbm.at[pl.ds(cbase, C)], vx_v, sem_l)
            pltpu.async_copy(vy_hbm.at[pl.ds(cbase, C)], vy_v, sem_l)
            pltpu.async_copy(vz_hbm.at[pl.ds(cbase, C)], vz_v, sem_l)

        def wait_chunk(j, buf):
            (idxu_v, idxv_v, rows_u, rows_v, dist_v,
             vx_v, vy_v, vz_v, out_v, sem_g, sem_l, sem_o) = buf
            cbase = base + j * C
            pltpu.make_async_copy(table_hbm.at[idxu_v], rows_u, sem_g).wait()
            pltpu.make_async_copy(table_hbm.at[idxv_v], rows_v, sem_g).wait()
            pltpu.make_async_copy(dist_hbm.at[pl.ds(cbase, C)], dist_v, sem_l).wait()
            pltpu.make_async_copy(vx_hbm.at[pl.ds(cbase, C)], vx_v, sem_l).wait()
            pltpu.make_async_copy(vy_hbm.at[pl.ds(cbase, C)], vy_v, sem_l).wait()
            pltpu.make_async_copy(vz_hbm.at[pl.ds(cbase, C)], vz_v, sem_l).wait()

        def wait_out(j, buf):
            out_v, sem_o = buf[8], buf[11]
            cbase = base + j * C
            pltpu.make_async_copy(
                out_v, out_hbm.at[pl.ds(cbase, C)], sem_o).wait()

        def compute_chunk(j, buf):
            (idxu_v, idxv_v, rows_u, rows_v, dist_v,
             vx_v, vy_v, vz_v, out_v, sem_g, sem_l, sem_o) = buf
            cbase = base + j * C

            def vec_body(i, carry2):
                o = i * L
                ridx = jnp.arange(L, dtype=jnp.int32) + o
                c0 = jnp.zeros((L,), jnp.int32)
                c1 = c0 + 1
                c2 = c0 + 2
                c3 = c0 + 3
                d = dist_v[pl.ds(o, L)]
                vx = vx_v[pl.ds(o, L)]
                vy = vy_v[pl.ds(o, L)]
                vz = vz_v[pl.ds(o, L)]
                qu = plsc.load_gather(rows_u, [ridx, c0])
                dux = plsc.load_gather(rows_u, [ridx, c1])
                duy = plsc.load_gather(rows_u, [ridx, c2])
                duz = plsc.load_gather(rows_u, [ridx, c3])
                qv = plsc.load_gather(rows_v, [ridx, c0])
                dvx = plsc.load_gather(rows_v, [ridx, c1])
                dvy = plsc.load_gather(rows_v, [ridx, c2])
                dvz = plsc.load_gather(rows_v, [ridx, c3])
                e = _edge_energy(d, vx, vy, vz, qu, qv,
                                 dux, duy, duz, dvx, dvy, dvz)
                out_v[pl.ds(o, L)] = e
                return carry2

            lax.fori_loop(0, n_vecs, vec_body, 0)
            pltpu.async_copy(out_v, out_hbm.at[pl.ds(cbase, C)], sem_o)

        issue_chunk(0, bufs[0])

        def pair_body(g, carry):
            j0 = 2 * g
            issue_chunk(j0 + 1, bufs[1])
            wait_chunk(j0, bufs[0])

            @pl.when(g > 0)
            def _():
                wait_out(j0 - 2, bufs[0])

            compute_chunk(j0, bufs[0])

            @pl.when(g < n_pairs - 1)
            def _():
                issue_chunk(j0 + 2, bufs[0])

            wait_chunk(j0 + 1, bufs[1])

            @pl.when(g > 0)
            def _():
                wait_out(j0 - 1, bufs[1])

            compute_chunk(j0 + 1, bufs[1])
            return carry

        lax.fori_loop(0, n_pairs, pair_body, 0)
        wait_out(n_chunks - 2, bufs[0])
        wait_out(n_chunks - 1, bufs[1])

    return sc_kernel


def kernel(distances_uv, vectors_uv, atomic_charges, atomic_dipoles, idx_u, idx_v):
    E = distances_uv.shape[0]
    N = atomic_charges.shape[0]
    C = 2000
    table = jnp.concatenate([atomic_charges[:, None], atomic_dipoles], axis=1)
    vx = vectors_uv[:, 0]
    vy = vectors_uv[:, 1]
    vz = vectors_uv[:, 2]
    sc = _make_sc_kernel(E, N, C)
    return sc(distances_uv, vx, vy, vz, table,
              idx_u.astype(jnp.int32), idx_v.astype(jnp.int32))
